# baseline (speedup 1.0000x reference)
import functools

import jax
import jax.numpy as jnp
from jax import lax
from jax.experimental import pallas as pl
from jax.experimental.pallas import tpu as pltpu
from jax.experimental.pallas import tpu_sc as plsc

VOCAB = 1000
DIM = 256
DSPLIT = 4
DSUB = DIM // DSPLIT
CHUNK = 128
IDX_BLOCK = 32
NBUF = 2


@functools.cache
def _build(B):
    info = plsc.get_sparse_core_info()
    NC, NS = info.num_cores, info.num_subcores
    NW = NC * NS
    NG = NW // DSPLIT
    b_per_g = B // NG
    assert b_per_g * NG == B and b_per_g % (CHUNK * IDX_BLOCK) == 0
    n_blocks = b_per_g // (CHUNK * IDX_BLOCK)
    n_chunks = b_per_g // CHUNK
    mesh = plsc.VectorSubcoreMesh(core_axis_name="c", subcore_axis_name="s")

    @functools.partial(
        pl.kernel,
        mesh=mesh,
        out_type=jax.ShapeDtypeStruct((B, DIM), jnp.float32),
        compiler_params=pltpu.CompilerParams(use_tc_tiling_on_sc=False,
                                             needs_layout_passes=False),
        scratch_types=[
            pltpu.VMEM((VOCAB, DSUB), jnp.float32),
            pltpu.VMEM((3, IDX_BLOCK * CHUNK), jnp.int32),
            pltpu.VMEM((NBUF * CHUNK, DSUB), jnp.float32),
            pltpu.SemaphoreType.DMA((3,)),
            pltpu.SemaphoreType.DMA((NBUF,)),
            pltpu.SemaphoreType.DMA,
        ],
    )
    def lookup(table_hbm, idx_hbm, out_hbm, tab_v, idx_v, rows_v,
               isem, wsem, tsem):
        wid = lax.axis_index("s") * NC + lax.axis_index("c")
        grp = wid // DSPLIT
        dpart = wid % DSPLIT
        d0 = dpart * DSUB
        base = grp * b_per_g

        def stage(ob, slot):
            pltpu.async_copy(idx_hbm.at[grp, ob], idx_v.at[slot],
                             isem.at[slot])

        def wait_idx(slot):
            pltpu.make_async_copy(idx_hbm.at[0, 0], idx_v.at[slot],
                                  isem.at[slot]).wait()

        def fire_write(pos, buf):
            pltpu.async_copy(rows_v.at[pl.ds(buf * CHUNK, CHUNK)],
                             out_hbm.at[pl.ds(pos, CHUNK), pl.ds(d0, DSUB)],
                             wsem.at[buf])

        def wait_write(buf):
            pltpu.make_async_copy(rows_v.at[pl.ds(0, CHUNK)],
                                  out_hbm.at[pl.ds(0, CHUNK),
                                             pl.ds(0, DSUB)],
                                  wsem.at[buf]).wait()

        pltpu.async_copy(table_hbm.at[:, pl.ds(d0, DSUB)], tab_v, tsem)
        stage(0, 0)
        stage(1, 1)
        stage(2, 2)
        pltpu.make_async_copy(table_hbm.at[:, pl.ds(0, DSUB)], tab_v,
                              tsem).wait()

        def chunk_body(g, carry):
            slot = (g // IDX_BLOCK) % 3
            j = g % IDX_BLOCK
            buf = g % NBUF
            rbase = buf * CHUNK

            @pl.when(g >= NBUF)
            def _():
                wait_write(buf)

            nk = DSUB // 16
            for q in range(CHUNK // 16):
                iv = idx_v[slot, pl.ds(j * CHUNK + 16 * q, 16)]
                rows = [iv[l] for l in range(16)]
                prev = None
                for l in range(16):
                    vals = [tab_v[rows[l], pl.ds(16 * k, 16)]
                            for k in range(nk)]
                    if prev is not None:
                        pu, pvals = prev
                        for k in range(nk):
                            rows_v[pu, pl.ds(16 * k, 16)] = pvals[k]
                    prev = (rbase + 16 * q + l, vals)
                pu, pvals = prev
                for k in range(nk):
                    rows_v[pu, pl.ds(16 * k, 16)] = pvals[k]
            fire_write(base + g * CHUNK, buf)

            @pl.when(j == IDX_BLOCK - 1)
            def _():
                ob = g // IDX_BLOCK

                @pl.when(ob + 3 < n_blocks)
                def _():
                    stage(ob + 3, ob % 3)

                @pl.when(ob + 1 < n_blocks)
                def _():
                    wait_idx((ob + 1) % 3)

            return carry

        wait_idx(0)
        lax.fori_loop(0, n_chunks, chunk_body, 0, unroll=False)

        for k in range(NBUF):
            wait_write((n_chunks - 1 - k) % NBUF)

    def run(table, idx_flat):
        idx3 = idx_flat.reshape(NG, n_blocks, IDX_BLOCK * CHUNK)
        return lookup(table, idx3)

    return run


def kernel(x, emb_weight):
    b, s = x.shape
    idx = x.reshape(-1).astype(jnp.int32)
    out = _build(idx.shape[0])(emb_weight, idx)
    return out.reshape(b, s, DIM)


# speedup vs baseline: 1.8091x; 1.8091x over previous
import functools

import jax
import jax.numpy as jnp
from jax import lax
from jax.experimental import pallas as pl
from jax.experimental.pallas import tpu as pltpu
from jax.experimental.pallas import tpu_sc as plsc

VOCAB = 1000
DIM = 256
CHUNK = 128
IDX_BLOCK = 32
NBUF = 3


@functools.cache
def _build(B):
    info = plsc.get_sparse_core_info()
    NC, NS = info.num_cores, info.num_subcores
    NW = NC * NS
    b_per_w = B // NW
    assert b_per_w * NW == B and b_per_w % (CHUNK * IDX_BLOCK) == 0
    n_blocks = b_per_w // (CHUNK * IDX_BLOCK)
    n_chunks = b_per_w // CHUNK
    assert n_blocks >= 3
    mesh = plsc.VectorSubcoreMesh(core_axis_name="c", subcore_axis_name="s")

    @functools.partial(
        pl.kernel,
        mesh=mesh,
        out_type=jax.ShapeDtypeStruct((B, DIM), jnp.float32),
        scratch_types=[
            pltpu.VMEM((3, IDX_BLOCK, CHUNK), jnp.int32),
            pltpu.VMEM((NBUF, CHUNK, DIM), jnp.float32),
            pltpu.SemaphoreType.DMA((3,)),
            pltpu.SemaphoreType.DMA((NBUF,)),
            pltpu.SemaphoreType.DMA((NBUF,)),
        ],
    )
    def lookup(table_hbm, idx_hbm, out_hbm, idx_v, rows_v, isem, gsem, wsem):
        wid = lax.axis_index("s") * NC + lax.axis_index("c")
        base = wid * b_per_w

        def stage(ob, slot):
            pltpu.async_copy(idx_hbm.at[wid, ob], idx_v.at[slot],
                             isem.at[slot])

        def wait_idx(slot):
            pltpu.make_async_copy(idx_hbm.at[wid, 0], idx_v.at[slot],
                                  isem.at[slot]).wait()

        def fire_gather(slot, j, buf):
            pltpu.async_copy(table_hbm.at[idx_v.at[slot, j]],
                             rows_v.at[buf], gsem.at[buf])

        def wait_gather(buf):
            pltpu.make_async_copy(out_hbm.at[pl.ds(0, CHUNK)],
                                  rows_v.at[buf], gsem.at[buf]).wait()

        def fire_write(pos, buf):
            pltpu.async_copy(rows_v.at[buf], out_hbm.at[pl.ds(pos, CHUNK)],
                             wsem.at[buf])

        def wait_write(buf):
            pltpu.make_async_copy(rows_v.at[buf],
                                  out_hbm.at[pl.ds(0, CHUNK)],
                                  wsem.at[buf]).wait()

        stage(0, 0)
        stage(1, 1)
        stage(2, 2)
        wait_idx(0)
        for j in range(IDX_BLOCK):
            buf = j % NBUF
            if j >= NBUF:
                wait_write(buf)
            fire_gather(0, j, buf)
            if j >= 1:
                pb = (j - 1) % NBUF
                wait_gather(pb)
                fire_write(base + (j - 1) * CHUNK, pb)

        def block(ob, carry):
            slot = ob % 3
            wait_idx(slot)
            g0 = ob * IDX_BLOCK
            for j in range(IDX_BLOCK):
                buf = (g0 + j) % NBUF
                wait_write(buf)
                fire_gather(slot, j, buf)
                pb = (g0 + j - 1) % NBUF
                wait_gather(pb)
                fire_write(base + (g0 + j - 1) * CHUNK, pb)

            @pl.when(ob + 2 < n_blocks)
            def _():
                stage(ob + 2, (ob + 2) % 3)

            return carry

        lax.fori_loop(1, n_blocks, block, 0, unroll=False)

        last = n_chunks - 1
        lb = last % NBUF
        wait_gather(lb)
        fire_write(base + last * CHUNK, lb)
        for k in range(NBUF):
            wait_write((last - k) % NBUF)

    def run(table, idx_flat):
        idx4 = idx_flat.reshape(NW, n_blocks, IDX_BLOCK, CHUNK)
        return lookup(table, idx4)

    return run


def kernel(x, emb_weight):
    b, s = x.shape
    idx = x.reshape(-1).astype(jnp.int32)
    out = _build(idx.shape[0])(emb_weight, idx)
    return out.reshape(b, s, DIM)
